# baseline (speedup 1.0000x reference)
import jax
import jax.numpy as jnp
from jax.experimental import pallas as pl
from jax.experimental.pallas import tpu as pltpu

N = 10000
D = 128
H1 = 64
H2 = 32

BM = 400
BD = 400


def _k1_small(x_ref, w1_ref, wa1_ref, wa2_ref, wa3_ref,
              xw1_ref, mua_ref, logvara_ref):
    x = x_ref[...]
    xw1_ref[...] = jnp.dot(x, w1_ref[...], preferred_element_type=jnp.float32)
    ha1 = jnp.tanh(jax.lax.dot_general(
        x, wa1_ref[...], (((0,), (0,)), ((), ())),
        preferred_element_type=jnp.float32))
    mua_ref[...] = jnp.dot(ha1, wa2_ref[...], preferred_element_type=jnp.float32)
    logvara_ref[...] = jnp.dot(ha1, wa3_ref[...], preferred_element_type=jnp.float32)


def _k2_pass1(adj_ref, xw1_ref, w23_ref, hw_ref):
    h1 = jnp.maximum(
        jnp.dot(adj_ref[...].astype(jnp.bfloat16),
                xw1_ref[...].astype(jnp.bfloat16),
                preferred_element_type=jnp.float32),
        0.0)
    hw_ref[...] = jnp.dot(h1, w23_ref[...], preferred_element_type=jnp.float32)


def _k3_pass2(adj_ref, hw_ref, mua_ref, mu_ref, logvar_ref, feat_ref):
    ml = jnp.dot(adj_ref[...].astype(jnp.bfloat16),
                 hw_ref[...].astype(jnp.bfloat16),
                 preferred_element_type=jnp.float32)
    mu = ml[:, :H2]
    mu_ref[...] = mu
    logvar_ref[...] = ml[:, H2:]
    feat_ref[...] = jax.lax.dot_general(
        mu, mua_ref[...], (((1,), (1,)), ((), ())),
        preferred_element_type=jnp.float32)


def _k4_decoder(zi_ref, zj_ref, out_ref):
    out_ref[...] = jax.lax.dot_general(
        zi_ref[...], zj_ref[...], (((1,), (1,)), ((), ())),
        preferred_element_type=jnp.float32)


def kernel(x, adj, W1, W2, W3, Wa1, Wa2, Wa3):
    f32 = jnp.float32

    xw1, mu_a, logvar_a = pl.pallas_call(
        _k1_small,
        out_shape=(
            jax.ShapeDtypeStruct((N, H1), f32),
            jax.ShapeDtypeStruct((D, H2), f32),
            jax.ShapeDtypeStruct((D, H2), f32),
        ),
    )(x, W1, Wa1, Wa2, Wa3)

    w23 = jnp.concatenate([W2, W3], axis=1)

    grid1 = N // BM
    hw = pl.pallas_call(
        _k2_pass1,
        grid=(grid1,),
        in_specs=[
            pl.BlockSpec((BM, N), lambda i: (i, 0)),
            pl.BlockSpec((N, H1), lambda i: (0, 0)),
            pl.BlockSpec((H1, 2 * H2), lambda i: (0, 0)),
        ],
        out_specs=pl.BlockSpec((BM, 2 * H2), lambda i: (i, 0)),
        out_shape=jax.ShapeDtypeStruct((N, 2 * H2), f32),
        compiler_params=pltpu.CompilerParams(
            dimension_semantics=("parallel",)),
    )(adj, xw1, w23)

    mu, logvar, features = pl.pallas_call(
        _k3_pass2,
        grid=(grid1,),
        in_specs=[
            pl.BlockSpec((BM, N), lambda i: (i, 0)),
            pl.BlockSpec((N, 2 * H2), lambda i: (0, 0)),
            pl.BlockSpec((D, H2), lambda i: (0, 0)),
        ],
        out_specs=(
            pl.BlockSpec((BM, H2), lambda i: (i, 0)),
            pl.BlockSpec((BM, H2), lambda i: (i, 0)),
            pl.BlockSpec((BM, D), lambda i: (i, 0)),
        ),
        out_shape=(
            jax.ShapeDtypeStruct((N, H2), f32),
            jax.ShapeDtypeStruct((N, H2), f32),
            jax.ShapeDtypeStruct((N, D), f32),
        ),
        compiler_params=pltpu.CompilerParams(
            dimension_semantics=("parallel",)),
    )(adj, hw, mu_a)

    gridd = N // BD
    adj_rec = pl.pallas_call(
        _k4_decoder,
        grid=(gridd,),
        in_specs=[
            pl.BlockSpec((BD, H2), lambda i: (i, 0)),
            pl.BlockSpec((N, H2), lambda i: (0, 0)),
        ],
        out_specs=pl.BlockSpec((BD, N), lambda i: (i, 0)),
        out_shape=jax.ShapeDtypeStruct((N, N), f32),
        compiler_params=pltpu.CompilerParams(
            dimension_semantics=("parallel",)),
    )(mu, mu)

    return (adj_rec, features, mu, logvar, mu_a, logvar_a)


# speedup vs baseline: 1.0100x; 1.0100x over previous
import jax
import jax.numpy as jnp
from jax.experimental import pallas as pl
from jax.experimental.pallas import tpu as pltpu

N = 10000
D = 128
H1 = 64
H2 = 32

BM = 200
NSTRIP = N // BM


def _k1_small(x_ref, w1_ref, wa1_ref, wa2_ref, wa3_ref,
              xw1_ref, mua_ref, logvara_ref):
    x = x_ref[...]
    xw1_ref[...] = jnp.dot(x, w1_ref[...], preferred_element_type=jnp.float32)
    ha1 = jnp.tanh(jax.lax.dot_general(
        x, wa1_ref[...], (((0,), (0,)), ((), ())),
        preferred_element_type=jnp.float32))
    mua_ref[...] = jnp.dot(ha1, wa2_ref[...], preferred_element_type=jnp.float32)
    logvara_ref[...] = jnp.dot(ha1, wa3_ref[...], preferred_element_type=jnp.float32)


def _k2_fused(adj_ref, xw1_ref, w23_ref, mua_ref,
              mu_ref, logvar_ref, feat_ref, adjrec_ref,
              hw_ref, muf_ref):
    s = pl.program_id(0)
    r = jax.lax.rem(s, NSTRIP)

    @pl.when(s < NSTRIP)
    def _phase0():
        h1 = jnp.maximum(
            jnp.dot(adj_ref[...], xw1_ref[...],
                    preferred_element_type=jnp.float32), 0.0)
        hw_ref[pl.ds(r * BM, BM), :] = jnp.dot(
            h1, w23_ref[...], preferred_element_type=jnp.float32)

    @pl.when(jnp.logical_and(s >= NSTRIP, s < 2 * NSTRIP))
    def _phase1():
        ml = jnp.dot(adj_ref[...], hw_ref[...],
                     preferred_element_type=jnp.float32)
        mu = ml[:, :H2]
        mu_ref[...] = mu
        logvar_ref[...] = ml[:, H2:]
        muf_ref[pl.ds(r * BM, BM), :] = mu
        feat_ref[...] = jax.lax.dot_general(
            mu, mua_ref[...], (((1,), (1,)), ((), ())),
            preferred_element_type=jnp.float32)

    @pl.when(s >= 2 * NSTRIP)
    def _phase2():
        zi = muf_ref[pl.ds(r * BM, BM), :]
        adjrec_ref[...] = jax.lax.dot_general(
            zi, muf_ref[...], (((1,), (1,)), ((), ())),
            preferred_element_type=jnp.float32)


def kernel(x, adj, W1, W2, W3, Wa1, Wa2, Wa3):
    f32 = jnp.float32

    xw1, mu_a, logvar_a = pl.pallas_call(
        _k1_small,
        out_shape=(
            jax.ShapeDtypeStruct((N, H1), f32),
            jax.ShapeDtypeStruct((D, H2), f32),
            jax.ShapeDtypeStruct((D, H2), f32),
        ),
    )(x, W1, Wa1, Wa2, Wa3)

    w23 = jnp.concatenate([W2, W3], axis=1)

    last = NSTRIP - 1
    mu, logvar, features, adj_rec = pl.pallas_call(
        _k2_fused,
        grid=(3 * NSTRIP,),
        in_specs=[
            pl.BlockSpec((BM, N),
                         lambda s: (jnp.where(s >= 2 * NSTRIP, last,
                                              jax.lax.rem(s, NSTRIP)), 0)),
            pl.BlockSpec((N, H1), lambda s: (0, 0)),
            pl.BlockSpec((H1, 2 * H2), lambda s: (0, 0)),
            pl.BlockSpec((D, H2), lambda s: (0, 0)),
        ],
        out_specs=(
            pl.BlockSpec((BM, H2),
                         lambda s: (jnp.clip(s - NSTRIP, 0, last), 0)),
            pl.BlockSpec((BM, H2),
                         lambda s: (jnp.clip(s - NSTRIP, 0, last), 0)),
            pl.BlockSpec((BM, D),
                         lambda s: (jnp.clip(s - NSTRIP, 0, last), 0)),
            pl.BlockSpec((BM, N),
                         lambda s: (jnp.clip(s - 2 * NSTRIP, 0, last), 0)),
        ),
        out_shape=(
            jax.ShapeDtypeStruct((N, H2), f32),
            jax.ShapeDtypeStruct((N, H2), f32),
            jax.ShapeDtypeStruct((N, D), f32),
            jax.ShapeDtypeStruct((N, N), f32),
        ),
        scratch_shapes=[
            pltpu.VMEM((N, 2 * H2), f32),
            pltpu.VMEM((N, H2), f32),
        ],
        compiler_params=pltpu.CompilerParams(
            dimension_semantics=("arbitrary",)),
    )(adj, xw1, w23, mu_a)

    return (adj_rec, features, mu, logvar, mu_a, logvar_a)
